# initial kernel scaffold (unmeasured)
import jax
import jax.numpy as jnp
from jax import lax
from jax.experimental import pallas as pl
from jax.experimental.pallas import tpu as pltpu

N_DEV = 16
B, SQ, SKV = 2, 512, 512
HQ_G, DH = 128, 64
H_LOC = HQ_G // N_DEV
D_MODEL = 768
D_LOC = H_LOC * DH
ROWS = B * SQ
CHUNK = ROWS // N_DEV
WINDOW = 128
SCALE = 0.125


def kernel(x, Wq, K_ext, V_ext, Wo):
    def body(x_ref, wq_hbm, k_ref, v_ref, wo_hbm, out_ref,
             wq_s, wo_s, acc, rbuf, local_sems,
             rs_send, rs_recv, ag_send, ag_recv):
        m = lax.axis_index("i")
        left = lax.rem(m - 1 + N_DEV, N_DEV)
        right = lax.rem(m + 1, N_DEV)

        col0 = m * D_LOC
        cp_q = pltpu.make_async_copy(
            wq_hbm.at[:, pl.ds(col0, D_LOC)], wq_s, local_sems.at[0])
        cp_o = pltpu.make_async_copy(
            wo_hbm.at[pl.ds(col0, D_LOC), :], wo_s, local_sems.at[1])
        cp_q.start()
        cp_o.start()
        cp_q.wait()
        cp_o.wait()

        qi = lax.broadcasted_iota(jnp.int32, (SQ, SKV), 0)
        ki = lax.broadcasted_iota(jnp.int32, (SQ, SKV), 1)
        mask = jnp.abs(qi - ki) <= WINDOW

        for b in range(B):
            xb = x_ref[b]
            qb = jnp.dot(xb, wq_s[...],
                         preferred_element_type=jnp.float32)
            accb = jnp.zeros((SQ, D_MODEL), jnp.float32)
            for h in range(H_LOC):
                q = qb[:, h * DH:(h + 1) * DH]
                k = k_ref[b, :, h, :]
                v = v_ref[b, :, h, :]
                s = lax.dot_general(
                    q, k, (((1,), (1,)), ((), ())),
                    preferred_element_type=jnp.float32) * SCALE
                s = jnp.where(mask, s, -1e9)
                s = s - jnp.max(s, axis=-1, keepdims=True)
                w = jnp.exp(s)
                w = w / jnp.sum(w, axis=-1, keepdims=True)
                ctx = jnp.dot(w, v,
                              preferred_element_type=jnp.float32)
                accb = accb + jnp.dot(
                    ctx, wo_s[h * DH:(h + 1) * DH, :],
                    preferred_element_type=jnp.float32)
            acc[pl.ds(b * SQ, SQ), :] = accb

        barrier = pltpu.get_barrier_semaphore()
        for nbr in (left, right):
            pl.semaphore_signal(barrier, inc=1, device_id=(nbr,),
                                device_id_type=pl.DeviceIdType.MESH)
        pl.semaphore_wait(barrier, 2)

        for s in range(N_DEV - 1):
            off_send = lax.rem(m - s + N_DEV, N_DEV) * CHUNK
            rdma = pltpu.make_async_remote_copy(
                src_ref=acc.at[pl.ds(off_send, CHUNK), :],
                dst_ref=rbuf.at[s],
                send_sem=rs_send.at[s],
                recv_sem=rs_recv.at[s],
                device_id=(right,),
                device_id_type=pl.DeviceIdType.MESH,
            )
            rdma.start()
            rdma.wait()
            off_recv = lax.rem(m - s - 1 + 2 * N_DEV, N_DEV) * CHUNK
            acc[pl.ds(off_recv, CHUNK), :] = (
                acc[pl.ds(off_recv, CHUNK), :] + rbuf[s])

        for s in range(N_DEV - 1):
            off = lax.rem(m + 1 - s + 2 * N_DEV, N_DEV) * CHUNK
            rdma = pltpu.make_async_remote_copy(
                src_ref=acc.at[pl.ds(off, CHUNK), :],
                dst_ref=acc.at[pl.ds(off, CHUNK), :],
                send_sem=ag_send.at[s],
                recv_sem=ag_recv.at[s],
                device_id=(right,),
                device_id_type=pl.DeviceIdType.MESH,
            )
            rdma.start()
            rdma.wait()

        for b in range(B):
            out_ref[b, :, :] = acc[b * SQ:(b + 1) * SQ, :]

    return pl.pallas_call(
        body,
        out_shape=jax.ShapeDtypeStruct((B, SQ, D_MODEL), jnp.float32),
        in_specs=[
            pl.BlockSpec(memory_space=pltpu.VMEM),
            pl.BlockSpec(memory_space=pltpu.ANY),
            pl.BlockSpec(memory_space=pltpu.VMEM),
            pl.BlockSpec(memory_space=pltpu.VMEM),
            pl.BlockSpec(memory_space=pltpu.ANY),
        ],
        out_specs=pl.BlockSpec(memory_space=pltpu.VMEM),
        scratch_shapes=[
            pltpu.VMEM((D_MODEL, D_LOC), jnp.float32),
            pltpu.VMEM((D_LOC, D_MODEL), jnp.float32),
            pltpu.VMEM((ROWS, D_MODEL), jnp.float32),
            pltpu.VMEM((N_DEV - 1, CHUNK, D_MODEL), jnp.float32),
            pltpu.SemaphoreType.DMA((2,)),
            pltpu.SemaphoreType.DMA((N_DEV - 1,)),
            pltpu.SemaphoreType.DMA((N_DEV - 1,)),
            pltpu.SemaphoreType.DMA((N_DEV - 1,)),
            pltpu.SemaphoreType.DMA((N_DEV - 1,)),
        ],
        compiler_params=pltpu.CompilerParams(collective_id=0),
    )(x, Wq, K_ext, V_ext, Wo)


# baseline (device time: 147430 ns/iter reference)
import jax
import jax.numpy as jnp
from jax import lax
from jax.experimental import pallas as pl
from jax.experimental.pallas import tpu as pltpu

N_DEV = 16
B, SQ, SKV = 2, 512, 512
HQ_G, DH = 128, 64
H_LOC = HQ_G // N_DEV
D_MODEL = 768
D_LOC = H_LOC * DH
ROWS = B * SQ
CHUNK = ROWS // N_DEV
WINDOW = 128
SCALE = 0.125


def kernel(x, Wq, K_ext, V_ext, Wo):
    def body(x_ref, wq_hbm, k_ref, v_ref, wo_hbm, out_ref,
             wq_s, wo_s, acc, rbuf, local_sems,
             rs_send, rs_recv, ag_send, ag_recv):
        m = lax.axis_index("i")
        left = lax.rem(m - 1 + N_DEV, N_DEV)
        right = lax.rem(m + 1, N_DEV)

        col0 = m * D_LOC
        cp_q = pltpu.make_async_copy(
            wq_hbm.at[:, pl.ds(col0, D_LOC)], wq_s, local_sems.at[0])
        cp_o = pltpu.make_async_copy(
            wo_hbm.at[pl.ds(col0, D_LOC), :], wo_s, local_sems.at[1])
        cp_q.start()
        cp_o.start()
        cp_q.wait()
        cp_o.wait()

        qi = lax.broadcasted_iota(jnp.int32, (SQ, SKV), 0)
        ki = lax.broadcasted_iota(jnp.int32, (SQ, SKV), 1)
        mask = jnp.abs(qi - ki) <= WINDOW

        for b in range(B):
            xb = x_ref[b]
            qb = jnp.dot(xb, wq_s[...],
                         preferred_element_type=jnp.float32)
            accb = jnp.zeros((SQ, D_MODEL), jnp.float32)
            for h in range(H_LOC):
                q = qb[:, h * DH:(h + 1) * DH]
                k = k_ref[b, :, h, :]
                v = v_ref[b, :, h, :]
                s = lax.dot_general(
                    q, k, (((1,), (1,)), ((), ())),
                    preferred_element_type=jnp.float32) * SCALE
                s = jnp.where(mask, s, -1e9)
                s = s - jnp.max(s, axis=-1, keepdims=True)
                w = jnp.exp(s)
                w = w / jnp.sum(w, axis=-1, keepdims=True)
                ctx = jnp.dot(w, v,
                              preferred_element_type=jnp.float32)
                accb = accb + jnp.dot(
                    ctx, wo_s[h * DH:(h + 1) * DH, :],
                    preferred_element_type=jnp.float32)
            acc[pl.ds(b * SQ, SQ), :] = accb

        barrier = pltpu.get_barrier_semaphore()
        for nbr in (left, right):
            pl.semaphore_signal(barrier, inc=1, device_id=(nbr,),
                                device_id_type=pl.DeviceIdType.MESH)
        pl.semaphore_wait(barrier, 2)

        for s in range(N_DEV - 1):
            off_send = lax.rem(m - s + N_DEV, N_DEV) * CHUNK
            rdma = pltpu.make_async_remote_copy(
                src_ref=acc.at[pl.ds(off_send, CHUNK), :],
                dst_ref=rbuf.at[s],
                send_sem=rs_send.at[s],
                recv_sem=rs_recv.at[s],
                device_id=(right,),
                device_id_type=pl.DeviceIdType.MESH,
            )
            rdma.start()
            rdma.wait()
            off_recv = lax.rem(m - s - 1 + 2 * N_DEV, N_DEV) * CHUNK
            acc[pl.ds(off_recv, CHUNK), :] = (
                acc[pl.ds(off_recv, CHUNK), :] + rbuf[s])

        for s in range(N_DEV - 1):
            off = lax.rem(m + 1 - s + 2 * N_DEV, N_DEV) * CHUNK
            rdma = pltpu.make_async_remote_copy(
                src_ref=acc.at[pl.ds(off, CHUNK), :],
                dst_ref=acc.at[pl.ds(off, CHUNK), :],
                send_sem=ag_send.at[s],
                recv_sem=ag_recv.at[s],
                device_id=(right,),
                device_id_type=pl.DeviceIdType.MESH,
            )
            rdma.start()
            rdma.wait()

        for b in range(B):
            out_ref[b, :, :] = acc[b * SQ:(b + 1) * SQ, :]

    return pl.pallas_call(
        body,
        out_shape=jax.ShapeDtypeStruct((B, SQ, D_MODEL), jnp.float32),
        in_specs=[
            pl.BlockSpec(memory_space=pltpu.MemorySpace.VMEM),
            pl.BlockSpec(memory_space=pl.ANY),
            pl.BlockSpec(memory_space=pltpu.MemorySpace.VMEM),
            pl.BlockSpec(memory_space=pltpu.MemorySpace.VMEM),
            pl.BlockSpec(memory_space=pl.ANY),
        ],
        out_specs=pl.BlockSpec(memory_space=pltpu.MemorySpace.VMEM),
        scratch_shapes=[
            pltpu.VMEM((D_MODEL, D_LOC), jnp.float32),
            pltpu.VMEM((D_LOC, D_MODEL), jnp.float32),
            pltpu.VMEM((ROWS, D_MODEL), jnp.float32),
            pltpu.VMEM((N_DEV - 1, CHUNK, D_MODEL), jnp.float32),
            pltpu.SemaphoreType.DMA((2,)),
            pltpu.SemaphoreType.DMA((N_DEV - 1,)),
            pltpu.SemaphoreType.DMA((N_DEV - 1,)),
            pltpu.SemaphoreType.DMA((N_DEV - 1,)),
            pltpu.SemaphoreType.DMA((N_DEV - 1,)),
        ],
        compiler_params=pltpu.CompilerParams(collective_id=0),
    )(x, Wq, K_ext, V_ext, Wo)


# device time: 74395 ns/iter; 1.9817x vs baseline; 1.9817x over previous
import jax
import jax.numpy as jnp
from jax import lax
from jax.experimental import pallas as pl
from jax.experimental.pallas import tpu as pltpu

N_DEV = 16
B, SQ, SKV = 2, 512, 512
HQ_G, DH = 128, 64
H_LOC = HQ_G // N_DEV
D_MODEL = 768
D_LOC = H_LOC * DH
ROWS = B * SQ
WINDOW = 128
SCALE = 0.125

COMM = jnp.bfloat16

BIT_ORDER = (0, 2, 1, 3)
HALF = (512, 256, 128, 64)
RS_ROFF = (0, 512, 768, 896)
AG_ROFF = (0, 64, 192, 448)


def kernel(x, Wq, K_ext, V_ext, Wo):
    K2 = K_ext.reshape(B, SKV, H_LOC * DH)
    V2 = V_ext.reshape(B, SKV, H_LOC * DH)

    def body(x_ref, wq_hbm, k_ref, v_ref, wo_hbm, out_ref,
             wq_s, wo_s, acc, sbuf, rs_rbuf, ag_rbuf, local_sems,
             rs_send, rs_recv, ag_send, ag_recv):
        m = lax.axis_index("i")

        col0 = m * D_LOC
        cp_q = pltpu.make_async_copy(
            wq_hbm.at[:, pl.ds(col0, D_LOC)], wq_s, local_sems.at[0])
        cp_o = pltpu.make_async_copy(
            wo_hbm.at[pl.ds(col0, D_LOC), :], wo_s, local_sems.at[1])
        cp_q.start()
        cp_o.start()
        cp_q.wait()
        cp_o.wait()

        qi = lax.broadcasted_iota(jnp.int32, (SQ, SKV), 0)
        ki = lax.broadcasted_iota(jnp.int32, (SQ, SKV), 1)
        mask = jnp.abs(qi - ki) <= WINDOW

        for b in range(B):
            xb = x_ref[b]
            qb = jnp.dot(xb, wq_s[...],
                         preferred_element_type=jnp.float32)
            accb = jnp.zeros((SQ, D_MODEL), jnp.float32)
            for h in range(H_LOC):
                q = qb[:, h * DH:(h + 1) * DH]
                k = k_ref[b, :, h * DH:(h + 1) * DH]
                v = v_ref[b, :, h * DH:(h + 1) * DH]
                s = lax.dot_general(
                    q, k, (((1,), (1,)), ((), ())),
                    preferred_element_type=jnp.float32) * SCALE
                s = jnp.where(mask, s, -1e9)
                s = s - jnp.max(s, axis=-1, keepdims=True)
                w = jnp.exp(s)
                w = w / jnp.sum(w, axis=-1, keepdims=True)
                ctx = jnp.dot(w, v,
                              preferred_element_type=jnp.float32)
                accb = accb + jnp.dot(
                    ctx, wo_s[h * DH:(h + 1) * DH, :],
                    preferred_element_type=jnp.float32)
            acc[pl.ds(b * SQ, SQ), :] = accb

        barrier = pltpu.get_barrier_semaphore()
        for bit in BIT_ORDER:
            pl.semaphore_signal(barrier, inc=1, device_id=(m ^ (1 << bit),),
                                device_id_type=pl.DeviceIdType.MESH)
        pl.semaphore_wait(barrier, 4)

        off = 0
        sels = []
        for t in range(4):
            bit = BIT_ORDER[t]
            half = HALF[t]
            sel = (m >> bit) & 1
            partner = m ^ (1 << bit)
            send_off = off + (1 - sel) * half
            keep_off = off + sel * half
            sbuf[pl.ds(0, half), :] = acc[pl.ds(send_off, half), :].astype(COMM)
            rdma = pltpu.make_async_remote_copy(
                src_ref=sbuf.at[pl.ds(0, half), :],
                dst_ref=rs_rbuf.at[pl.ds(RS_ROFF[t], half), :],
                send_sem=rs_send.at[t],
                recv_sem=rs_recv.at[t],
                device_id=(partner,),
                device_id_type=pl.DeviceIdType.MESH,
            )
            rdma.start()
            rdma.wait()
            acc[pl.ds(keep_off, half), :] = (
                acc[pl.ds(keep_off, half), :]
                + rs_rbuf[pl.ds(RS_ROFF[t], half), :].astype(jnp.float32))
            off = keep_off
            sels.append(sel)

        for i, t in enumerate(reversed(range(4))):
            bit = BIT_ORDER[t]
            size = HALF[t]
            sel = sels[t]
            partner = m ^ (1 << bit)
            sbuf[pl.ds(0, size), :] = acc[pl.ds(off, size), :].astype(COMM)
            rdma = pltpu.make_async_remote_copy(
                src_ref=sbuf.at[pl.ds(0, size), :],
                dst_ref=ag_rbuf.at[pl.ds(AG_ROFF[i], size), :],
                send_sem=ag_send.at[i],
                recv_sem=ag_recv.at[i],
                device_id=(partner,),
                device_id_type=pl.DeviceIdType.MESH,
            )
            rdma.start()
            rdma.wait()
            p_off = off + (1 - 2 * sel) * size
            acc[pl.ds(p_off, size), :] = (
                ag_rbuf[pl.ds(AG_ROFF[i], size), :].astype(jnp.float32))
            off = off - sel * size

        for b in range(B):
            out_ref[b, :, :] = acc[b * SQ:(b + 1) * SQ, :]

    return pl.pallas_call(
        body,
        out_shape=jax.ShapeDtypeStruct((B, SQ, D_MODEL), jnp.float32),
        in_specs=[
            pl.BlockSpec(memory_space=pltpu.MemorySpace.VMEM),
            pl.BlockSpec(memory_space=pl.ANY),
            pl.BlockSpec(memory_space=pltpu.MemorySpace.VMEM),
            pl.BlockSpec(memory_space=pltpu.MemorySpace.VMEM),
            pl.BlockSpec(memory_space=pl.ANY),
        ],
        out_specs=pl.BlockSpec(memory_space=pltpu.MemorySpace.VMEM),
        scratch_shapes=[
            pltpu.VMEM((D_MODEL, D_LOC), jnp.float32),
            pltpu.VMEM((D_LOC, D_MODEL), jnp.float32),
            pltpu.VMEM((ROWS, D_MODEL), jnp.float32),
            pltpu.VMEM((512, D_MODEL), COMM),
            pltpu.VMEM((960, D_MODEL), COMM),
            pltpu.VMEM((960, D_MODEL), COMM),
            pltpu.SemaphoreType.DMA((2,)),
            pltpu.SemaphoreType.DMA((4,)),
            pltpu.SemaphoreType.DMA((4,)),
            pltpu.SemaphoreType.DMA((4,)),
            pltpu.SemaphoreType.DMA((4,)),
        ],
        compiler_params=pltpu.CompilerParams(collective_id=0),
    )(x, Wq, K2, V2, Wo)


# device time: 62017 ns/iter; 2.3773x vs baseline; 1.1996x over previous
import jax
import jax.numpy as jnp
from jax import lax
from jax.experimental import pallas as pl
from jax.experimental.pallas import tpu as pltpu

N_DEV = 16
B, SQ, SKV = 2, 512, 512
HQ_G, DH = 128, 64
H_LOC = HQ_G // N_DEV
D_MODEL = 768
D_LOC = H_LOC * DH
ROWS = B * SQ
WINDOW = 128
SCALE = 0.125

COMM = jnp.bfloat16
COLW = D_MODEL // 2

BIT_ORDER = (0, 2, 1, 3)
HALF = (512, 256, 128, 64)
RS_ROFF = (0, 512, 768, 896)
AG_ROFF = (960, 1024, 1152, 1408)


def kernel(x, Wq, K_ext, V_ext, Wo):
    x2 = x.reshape(ROWS, D_MODEL)
    K2 = K_ext.reshape(B * SKV, H_LOC * DH)
    V2 = V_ext.reshape(B * SKV, H_LOC * DH)

    def body(x_ref, wq_hbm, k_ref, v_ref, wo_hbm, out_ref,
             wq_s, wo_s, acc, sbuf, rbuf, local_sems,
             sa_send, sa_recv, sb_send, sb_recv):
        m = lax.axis_index("i")

        barrier = pltpu.get_barrier_semaphore()
        for bit in BIT_ORDER:
            pl.semaphore_signal(barrier, inc=1, device_id=(m ^ (1 << bit),),
                                device_id_type=pl.DeviceIdType.MESH)

        col0 = m * D_LOC
        cp_q = pltpu.make_async_copy(
            wq_hbm.at[:, pl.ds(col0, D_LOC)], wq_s, local_sems.at[0])
        cp_o = pltpu.make_async_copy(
            wo_hbm.at[pl.ds(col0, D_LOC), :], wo_s, local_sems.at[1])
        cp_q.start()
        cp_o.start()
        cp_q.wait()
        cp_o.wait()

        off = 0
        sels = []
        rs_plan = []
        for t in range(4):
            bit = BIT_ORDER[t]
            half = HALF[t]
            sel = (m >> bit) & 1
            partner = m ^ (1 << bit)
            send_off = off + (1 - sel) * half
            keep_off = off + sel * half
            rs_plan.append((half, send_off, keep_off, RS_ROFF[t], partner))
            off = keep_off
            sels.append(sel)
        ag_plan = []
        for i, t in enumerate(reversed(range(4))):
            bit = BIT_ORDER[t]
            size = HALF[t]
            sel = sels[t]
            partner = m ^ (1 << bit)
            p_off = off + (1 - 2 * sel) * size
            ag_plan.append((size, off, p_off, AG_ROFF[i], partner))
            off = off - sel * size

        qi = lax.broadcasted_iota(jnp.int32, (SQ, SKV), 0)
        ki = lax.broadcasted_iota(jnp.int32, (SQ, SKV), 1)
        mask = jnp.abs(qi - ki) <= WINDOW

        def compute_batch(b):
            xb = x_ref[pl.ds(b * SQ, SQ), :]
            qb = jnp.dot(xb, wq_s[...],
                         preferred_element_type=jnp.float32)
            accb = jnp.zeros((SQ, D_MODEL), jnp.float32)
            for h in range(H_LOC):
                q = qb[:, h * DH:(h + 1) * DH]
                k = k_ref[pl.ds(b * SKV, SKV), pl.ds(h * DH, DH)]
                v = v_ref[pl.ds(b * SKV, SKV), pl.ds(h * DH, DH)]
                s = lax.dot_general(
                    q, k, (((1,), (1,)), ((), ())),
                    preferred_element_type=jnp.float32) * SCALE
                s = jnp.where(mask, s, -1e9)
                s = s - jnp.max(s, axis=-1, keepdims=True)
                w = jnp.exp(s)
                w = w / jnp.sum(w, axis=-1, keepdims=True)
                ctx = jnp.dot(w, v,
                              preferred_element_type=jnp.float32)
                accb = accb + jnp.dot(
                    ctx, wo_s[h * DH:(h + 1) * DH, :],
                    preferred_element_type=jnp.float32)
            acc[pl.ds(b * SQ, SQ), :] = accb

        def stage_send(k, c0, send_sems, recv_sems):
            if k < 4:
                half, src_off, _, r_off, partner = rs_plan[k]
            else:
                size, src_off, _, r_off, partner = ag_plan[k - 4]
                half = size
            sbuf[pl.ds(0, half), pl.ds(c0, COLW)] = (
                acc[pl.ds(src_off, half), pl.ds(c0, COLW)].astype(COMM))
            rdma = pltpu.make_async_remote_copy(
                src_ref=sbuf.at[pl.ds(0, half), pl.ds(c0, COLW)],
                dst_ref=rbuf.at[pl.ds(r_off, half), pl.ds(c0, COLW)],
                send_sem=send_sems.at[k],
                recv_sem=recv_sems.at[k],
                device_id=(partner,),
                device_id_type=pl.DeviceIdType.MESH,
            )
            rdma.start()
            return rdma

        def process(k, c0):
            if k < 4:
                half, _, keep_off, r_off, _ = rs_plan[k]
                acc[pl.ds(keep_off, half), pl.ds(c0, COLW)] = (
                    acc[pl.ds(keep_off, half), pl.ds(c0, COLW)]
                    + rbuf[pl.ds(r_off, half), pl.ds(c0, COLW)].astype(
                        jnp.float32))
            else:
                size, _, p_off, r_off, _ = ag_plan[k - 4]
                acc[pl.ds(p_off, size), pl.ds(c0, COLW)] = (
                    rbuf[pl.ds(r_off, size), pl.ds(c0, COLW)].astype(
                        jnp.float32))

        b_send = 1 - sels[0]
        compute_batch(b_send)
        pl.semaphore_wait(barrier, 4)
        ra = [None] * 8
        rb = [None] * 8
        ra[0] = stage_send(0, 0, sa_send, sa_recv)
        rb[0] = stage_send(0, COLW, sb_send, sb_recv)
        compute_batch(sels[0])

        for k in range(8):
            ra[k].wait()
            process(k, 0)
            if k < 7:
                ra[k + 1] = stage_send(k + 1, 0, sa_send, sa_recv)
            rb[k].wait()
            process(k, COLW)
            if k < 7:
                rb[k + 1] = stage_send(k + 1, COLW, sb_send, sb_recv)

        for b in range(B):
            out_ref[b, :, :] = acc[b * SQ:(b + 1) * SQ, :]

    return pl.pallas_call(
        body,
        out_shape=jax.ShapeDtypeStruct((B, SQ, D_MODEL), jnp.float32),
        in_specs=[
            pl.BlockSpec(memory_space=pltpu.MemorySpace.VMEM),
            pl.BlockSpec(memory_space=pl.ANY),
            pl.BlockSpec(memory_space=pltpu.MemorySpace.VMEM),
            pl.BlockSpec(memory_space=pltpu.MemorySpace.VMEM),
            pl.BlockSpec(memory_space=pl.ANY),
        ],
        out_specs=pl.BlockSpec(memory_space=pltpu.MemorySpace.VMEM),
        scratch_shapes=[
            pltpu.VMEM((D_MODEL, D_LOC), jnp.float32),
            pltpu.VMEM((D_LOC, D_MODEL), jnp.float32),
            pltpu.VMEM((ROWS, D_MODEL), jnp.float32),
            pltpu.VMEM((512, D_MODEL), COMM),
            pltpu.VMEM((1920, D_MODEL), COMM),
            pltpu.SemaphoreType.DMA((2,)),
            pltpu.SemaphoreType.DMA((8,)),
            pltpu.SemaphoreType.DMA((8,)),
            pltpu.SemaphoreType.DMA((8,)),
            pltpu.SemaphoreType.DMA((8,)),
        ],
        compiler_params=pltpu.CompilerParams(collective_id=0),
    )(x2, Wq, K2, V2, Wo)


# device time: 54922 ns/iter; 2.6844x vs baseline; 1.1292x over previous
import jax
import jax.numpy as jnp
from jax import lax
from jax.experimental import pallas as pl
from jax.experimental.pallas import tpu as pltpu

N_DEV = 16
B, SQ, SKV = 2, 512, 512
HQ_G, DH = 128, 64
H_LOC = HQ_G // N_DEV
D_MODEL = 768
D_LOC = H_LOC * DH
ROWS = B * SQ
WINDOW = 128
SCALE = 0.125

COMM = jnp.bfloat16
COLW = D_MODEL // 2

STEPS = (
    ("rs", 0, 256, 0),
    ("rs", 2, 64, 768),
    ("ag", 2, 64, 960),
    ("ag", 0, 256, 1152),
)


def kernel(x, Wq, K_ext, V_ext, Wo):
    x2 = x.reshape(ROWS, D_MODEL)
    K2 = K_ext.reshape(B * SKV, H_LOC * DH)
    V2 = V_ext.reshape(B * SKV, H_LOC * DH)

    def body(x_ref, wq_hbm, k_ref, v_ref, wo_hbm, out_ref,
             wq_s, wo_s, acc, sbuf, rbuf, local_sems,
             sa_send, sa_recv, sb_send, sb_recv):
        m = lax.axis_index("i")
        v0 = m & 3
        v1 = (m >> 2) & 3
        base1 = v0 * 256
        keep1 = base1 + v1 * 64

        barrier = pltpu.get_barrier_semaphore()
        for shift in (0, 2):
            for d in (1, 2, 3):
                pl.semaphore_signal(
                    barrier, inc=1, device_id=(m ^ (d << shift),),
                    device_id_type=pl.DeviceIdType.MESH)

        col0 = m * D_LOC
        cp_q = pltpu.make_async_copy(
            wq_hbm.at[:, pl.ds(col0, D_LOC)], wq_s, local_sems.at[0])
        cp_o = pltpu.make_async_copy(
            wo_hbm.at[pl.ds(col0, D_LOC), :], wo_s, local_sems.at[1])
        cp_q.start()
        cp_o.start()
        cp_q.wait()
        cp_o.wait()

        qi = lax.broadcasted_iota(jnp.int32, (SQ, SKV), 0)
        ki = lax.broadcasted_iota(jnp.int32, (SQ, SKV), 1)
        mask = jnp.abs(qi - ki) <= WINDOW

        def compute_batch(b):
            xb = x_ref[pl.ds(b * SQ, SQ), :]
            qb = jnp.dot(xb, wq_s[...],
                         preferred_element_type=jnp.float32)
            accb = jnp.zeros((SQ, D_MODEL), jnp.float32)
            for h in range(H_LOC):
                q = qb[:, h * DH:(h + 1) * DH]
                k = k_ref[pl.ds(b * SKV, SKV), pl.ds(h * DH, DH)]
                v = v_ref[pl.ds(b * SKV, SKV), pl.ds(h * DH, DH)]
                s = lax.dot_general(
                    q, k, (((1,), (1,)), ((), ())),
                    preferred_element_type=jnp.float32) * SCALE
                s = jnp.where(mask, s, -1e9)
                s = s - jnp.max(s, axis=-1, keepdims=True)
                w = jnp.exp(s)
                w = w / jnp.sum(w, axis=-1, keepdims=True)
                ctx = jnp.dot(w, v,
                              preferred_element_type=jnp.float32)
                accb = accb + jnp.dot(
                    ctx, wo_s[h * DH:(h + 1) * DH, :],
                    preferred_element_type=jnp.float32)
            acc[pl.ds(b * SQ, SQ), :] = accb

        def step_params(k):
            kind, shift, size, roff = STEPS[k]
            vlev = v0 if shift == 0 else v1
            base = 0 if shift == 0 else base1
            return kind, shift, size, roff, vlev, base

        def send_one(k, d, c0, send_sems, recv_sems):
            kind, shift, size, roff, vlev, base = step_params(k)
            if kind == "rs":
                srow = base + (vlev ^ d) * size
                slot = (d - 1) * size
                sbuf[pl.ds(slot, size), pl.ds(c0, COLW)] = (
                    acc[pl.ds(srow, size), pl.ds(c0, COLW)].astype(COMM))
                src = sbuf.at[pl.ds(slot, size), pl.ds(c0, COLW)]
            else:
                if d == 1:
                    my_off = keep1 if shift == 2 else base1
                    sbuf[pl.ds(0, size), pl.ds(c0, COLW)] = (
                        acc[pl.ds(my_off, size), pl.ds(c0, COLW)].astype(COMM))
                src = sbuf.at[pl.ds(0, size), pl.ds(c0, COLW)]
            rdma = pltpu.make_async_remote_copy(
                src_ref=src,
                dst_ref=rbuf.at[pl.ds(roff + (d - 1) * size, size),
                                pl.ds(c0, COLW)],
                send_sem=send_sems.at[k * 3 + d - 1],
                recv_sem=recv_sems.at[k * 3 + d - 1],
                device_id=(m ^ (d << shift),),
                device_id_type=pl.DeviceIdType.MESH,
            )
            rdma.start()
            return rdma

        def send_step(k, c0, send_sems, recv_sems, ds=(1, 2, 3)):
            return [send_one(k, d, c0, send_sems, recv_sems) for d in ds]

        def process(k, c0):
            kind, shift, size, roff, vlev, base = step_params(k)
            if kind == "rs":
                keep = base + vlev * size
                total = acc[pl.ds(keep, size), pl.ds(c0, COLW)]
                for j in range(3):
                    total = total + rbuf[
                        pl.ds(roff + j * size, size),
                        pl.ds(c0, COLW)].astype(jnp.float32)
                acc[pl.ds(keep, size), pl.ds(c0, COLW)] = total
            else:
                for d in (1, 2, 3):
                    acc[pl.ds(base + (vlev ^ d) * size, size),
                        pl.ds(c0, COLW)] = rbuf[
                        pl.ds(roff + (d - 1) * size, size),
                        pl.ds(c0, COLW)].astype(jnp.float32)

        b_other = 1 - (v0 >> 1)
        compute_batch(b_other)
        pl.semaphore_wait(barrier, 6)
        ra = [None] * 4
        rb = [None] * 4
        ra[0] = send_step(0, 0, sa_send, sa_recv, ds=(2, 3))
        rb[0] = send_step(0, COLW, sb_send, sb_recv, ds=(2, 3))
        compute_batch(1 - b_other)
        ra[0] += send_step(0, 0, sa_send, sa_recv, ds=(1,))
        rb[0] += send_step(0, COLW, sb_send, sb_recv, ds=(1,))

        for k in range(4):
            for r in ra[k]:
                r.wait()
            process(k, 0)
            if k < 3:
                ra[k + 1] = send_step(k + 1, 0, sa_send, sa_recv)
            for r in rb[k]:
                r.wait()
            process(k, COLW)
            if k < 3:
                rb[k + 1] = send_step(k + 1, COLW, sb_send, sb_recv)

        for b in range(B):
            out_ref[b, :, :] = acc[b * SQ:(b + 1) * SQ, :]

    return pl.pallas_call(
        body,
        out_shape=jax.ShapeDtypeStruct((B, SQ, D_MODEL), jnp.float32),
        in_specs=[
            pl.BlockSpec(memory_space=pltpu.MemorySpace.VMEM),
            pl.BlockSpec(memory_space=pl.ANY),
            pl.BlockSpec(memory_space=pltpu.MemorySpace.VMEM),
            pl.BlockSpec(memory_space=pltpu.MemorySpace.VMEM),
            pl.BlockSpec(memory_space=pl.ANY),
        ],
        out_specs=pl.BlockSpec(memory_space=pltpu.MemorySpace.VMEM),
        scratch_shapes=[
            pltpu.VMEM((D_MODEL, D_LOC), jnp.float32),
            pltpu.VMEM((D_LOC, D_MODEL), jnp.float32),
            pltpu.VMEM((ROWS, D_MODEL), jnp.float32),
            pltpu.VMEM((768, D_MODEL), COMM),
            pltpu.VMEM((1920, D_MODEL), COMM),
            pltpu.SemaphoreType.DMA((2,)),
            pltpu.SemaphoreType.DMA((12,)),
            pltpu.SemaphoreType.DMA((12,)),
            pltpu.SemaphoreType.DMA((12,)),
            pltpu.SemaphoreType.DMA((12,)),
        ],
        compiler_params=pltpu.CompilerParams(collective_id=0),
    )(x2, Wq, K2, V2, Wo)


# device time: 54799 ns/iter; 2.6904x vs baseline; 1.0022x over previous
import jax
import jax.numpy as jnp
from jax import lax
from jax.experimental import pallas as pl
from jax.experimental.pallas import tpu as pltpu

N_DEV = 16
B, SQ, SKV = 2, 512, 512
HQ_G, DH = 128, 64
H_LOC = HQ_G // N_DEV
D_MODEL = 768
D_LOC = H_LOC * DH
ROWS = B * SQ
WINDOW = 128
SCALE = 0.125

COMM = jnp.bfloat16
COLW = D_MODEL // 2

STEP_SIZES = (256, 64, 64, 256)
STEP_ROFF = (0, 768, 960, 1152)


def kernel(x, Wq, K_ext, V_ext, Wo):
    x2 = x.reshape(ROWS, D_MODEL)
    K2 = K_ext.reshape(B * SKV, H_LOC * DH)
    V2 = V_ext.reshape(B * SKV, H_LOC * DH)

    def body(x_ref, wq_hbm, k_ref, v_ref, wo_hbm, out_ref,
             wq_s, wo_s, acc, sbuf, rbuf, local_sems,
             sa_send, sa_recv, sb_send, sb_recv):
        m = lax.axis_index("i")

        barrier = pltpu.get_barrier_semaphore()
        for shift in (0, 2):
            for d in (1, 2, 3):
                pl.semaphore_signal(
                    barrier, inc=1, device_id=(m ^ (d << shift),),
                    device_id_type=pl.DeviceIdType.MESH)

        col0 = m * D_LOC
        cp_q = pltpu.make_async_copy(
            wq_hbm.at[:, pl.ds(col0, D_LOC)], wq_s, local_sems.at[0])
        cp_o = pltpu.make_async_copy(
            wo_hbm.at[pl.ds(col0, D_LOC), :], wo_s, local_sems.at[1])
        cp_q.start()
        cp_o.start()
        cp_q.wait()
        cp_o.wait()

        qi = lax.broadcasted_iota(jnp.int32, (SQ, SKV), 0)
        ki = lax.broadcasted_iota(jnp.int32, (SQ, SKV), 1)
        mask = jnp.abs(qi - ki) <= WINDOW

        def compute_batch(b):
            xb = x_ref[pl.ds(b * SQ, SQ), :]
            qb = jnp.dot(xb, wq_s[...],
                         preferred_element_type=jnp.float32)
            accb = jnp.zeros((SQ, D_MODEL), jnp.float32)
            for h in range(H_LOC):
                q = qb[:, h * DH:(h + 1) * DH]
                k = k_ref[pl.ds(b * SKV, SKV), pl.ds(h * DH, DH)]
                v = v_ref[pl.ds(b * SKV, SKV), pl.ds(h * DH, DH)]
                s = lax.dot_general(
                    q, k, (((1,), (1,)), ((), ())),
                    preferred_element_type=jnp.float32) * SCALE
                w = jnp.exp(jnp.where(mask, s, -1e9))
                w = w / jnp.sum(w, axis=-1, keepdims=True)
                ctx = jnp.dot(w, v,
                              preferred_element_type=jnp.float32)
                accb = accb + jnp.dot(
                    ctx, wo_s[h * DH:(h + 1) * DH, :],
                    preferred_element_type=jnp.float32)
            acc[pl.ds(b * SQ, SQ), :] = accb

        class Chain:

            def __init__(self, c0, order, send_sems, recv_sems):
                self.c0 = c0
                self.send_sems = send_sems
                self.recv_sems = recv_sems
                s1, s2 = order
                v_first = (m >> s1) & 3
                v_second = (m >> s2) & 3
                base2 = v_first * 256
                keep = base2 + v_second * 64
                self.steps = (
                    ("rs", s1, v_first, 0, 0),
                    ("rs", s2, v_second, base2, base2),
                    ("ag", s2, v_second, base2, keep),
                    ("ag", s1, v_first, 0, base2),
                )

            def send_one(self, k, d):
                kind, shift, vlev, base, s_off = self.steps[k]
                size = STEP_SIZES[k]
                roff = STEP_ROFF[k]
                cs = pl.ds(self.c0, COLW)
                if kind == "rs":
                    srow = base + (vlev ^ d) * size
                    slot = (d - 1) * size
                    sbuf[pl.ds(slot, size), cs] = (
                        acc[pl.ds(srow, size), cs].astype(COMM))
                    src = sbuf.at[pl.ds(slot, size), cs]
                else:
                    if d == 1:
                        sbuf[pl.ds(0, size), cs] = (
                            acc[pl.ds(s_off, size), cs].astype(COMM))
                    src = sbuf.at[pl.ds(0, size), cs]
                rdma = pltpu.make_async_remote_copy(
                    src_ref=src,
                    dst_ref=rbuf.at[pl.ds(roff + (d - 1) * size, size), cs],
                    send_sem=self.send_sems.at[k * 3 + d - 1],
                    recv_sem=self.recv_sems.at[k * 3 + d - 1],
                    device_id=(m ^ (d << shift),),
                    device_id_type=pl.DeviceIdType.MESH,
                )
                rdma.start()
                return rdma

            def send_step(self, k, ds=(1, 2, 3)):
                return [self.send_one(k, d) for d in ds]

            def process(self, k):
                kind, shift, vlev, base, _ = self.steps[k]
                size = STEP_SIZES[k]
                roff = STEP_ROFF[k]
                cs = pl.ds(self.c0, COLW)
                if kind == "rs":
                    keep = base + vlev * size
                    total = acc[pl.ds(keep, size), cs]
                    for j in range(3):
                        total = total + rbuf[
                            pl.ds(roff + j * size, size), cs].astype(
                            jnp.float32)
                    acc[pl.ds(keep, size), cs] = total
                else:
                    for d in (1, 2, 3):
                        acc[pl.ds(base + (vlev ^ d) * size, size), cs] = (
                            rbuf[pl.ds(roff + (d - 1) * size, size),
                                 cs].astype(jnp.float32))

        ca = Chain(0, (0, 2), sa_send, sa_recv)
        cb = Chain(COLW, (2, 0), sb_send, sb_recv)

        v0 = m & 3
        b_other = 1 - (v0 >> 1)
        compute_batch(b_other)
        pl.semaphore_wait(barrier, 6)
        ra = [None] * 4
        rb = [None] * 4
        ra[0] = ca.send_step(0, ds=(2, 3))
        compute_batch(1 - b_other)
        ra[0] += ca.send_step(0, ds=(1,))
        rb[0] = cb.send_step(0)

        for k in range(4):
            for r in ra[k]:
                r.wait()
            ca.process(k)
            if k < 3:
                ra[k + 1] = ca.send_step(k + 1)
            for r in rb[k]:
                r.wait()
            cb.process(k)
            if k < 3:
                rb[k + 1] = cb.send_step(k + 1)

        for b in range(B):
            out_ref[b, :, :] = acc[b * SQ:(b + 1) * SQ, :]

    return pl.pallas_call(
        body,
        out_shape=jax.ShapeDtypeStruct((B, SQ, D_MODEL), jnp.float32),
        in_specs=[
            pl.BlockSpec(memory_space=pltpu.MemorySpace.VMEM),
            pl.BlockSpec(memory_space=pl.ANY),
            pl.BlockSpec(memory_space=pltpu.MemorySpace.VMEM),
            pl.BlockSpec(memory_space=pltpu.MemorySpace.VMEM),
            pl.BlockSpec(memory_space=pl.ANY),
        ],
        out_specs=pl.BlockSpec(memory_space=pltpu.MemorySpace.VMEM),
        scratch_shapes=[
            pltpu.VMEM((D_MODEL, D_LOC), jnp.float32),
            pltpu.VMEM((D_LOC, D_MODEL), jnp.float32),
            pltpu.VMEM((ROWS, D_MODEL), jnp.float32),
            pltpu.VMEM((768, D_MODEL), COMM),
            pltpu.VMEM((1920, D_MODEL), COMM),
            pltpu.SemaphoreType.DMA((2,)),
            pltpu.SemaphoreType.DMA((12,)),
            pltpu.SemaphoreType.DMA((12,)),
            pltpu.SemaphoreType.DMA((12,)),
            pltpu.SemaphoreType.DMA((12,)),
        ],
        compiler_params=pltpu.CompilerParams(collective_id=0),
    )(x2, Wq, K2, V2, Wo)


# device time: 54078 ns/iter; 2.7262x vs baseline; 1.0133x over previous
import jax
import jax.numpy as jnp
from jax import lax
from jax.experimental import pallas as pl
from jax.experimental.pallas import tpu as pltpu

N_DEV = 16
B, SQ, SKV = 2, 512, 512
HQ_G, DH = 128, 64
H_LOC = HQ_G // N_DEV
D_MODEL = 768
D_LOC = H_LOC * DH
ROWS = B * SQ
WINDOW = 128
SCALE = 0.125

COMM = jnp.bfloat16
COLW = D_MODEL // 2

STEP_SIZES = (256, 64, 64, 256)
STEP_ROFF = (0, 768, 960, 1152)


def kernel(x, Wq, K_ext, V_ext, Wo):
    x2 = x.reshape(ROWS, D_MODEL)
    K2 = K_ext.reshape(B * SKV, H_LOC * DH)
    V2 = V_ext.reshape(B * SKV, H_LOC * DH)

    def body(x_ref, wq_hbm, k_ref, v_ref, wo_hbm, out_ref,
             wq_s, wo_s, acc, sbuf, rbuf, local_sems,
             sa_send, sa_recv, sb_send, sb_recv):
        m = lax.axis_index("i")

        barrier = pltpu.get_barrier_semaphore()
        for shift in (0, 2):
            for d in (1, 2, 3):
                pl.semaphore_signal(
                    barrier, inc=1, device_id=(m ^ (d << shift),),
                    device_id_type=pl.DeviceIdType.MESH)

        col0 = m * D_LOC
        cp_q = pltpu.make_async_copy(
            wq_hbm.at[:, pl.ds(col0, D_LOC)], wq_s, local_sems.at[0])
        cp_o = pltpu.make_async_copy(
            wo_hbm.at[pl.ds(col0, D_LOC), :], wo_s, local_sems.at[1])
        cp_q.start()
        cp_o.start()
        cp_q.wait()
        cp_o.wait()

        qi = lax.broadcasted_iota(jnp.int32, (SQ, SKV), 0)
        ki = lax.broadcasted_iota(jnp.int32, (SQ, SKV), 1)
        mask = jnp.abs(qi - ki) <= WINDOW

        def compute_batch(b):
            xb = x_ref[pl.ds(b * SQ, SQ), :]
            qb = jnp.dot(xb, wq_s[...],
                         preferred_element_type=jnp.float32)
            accb = jnp.zeros((SQ, D_MODEL), jnp.float32)
            for h in range(H_LOC):
                q = qb[:, h * DH:(h + 1) * DH]
                k = k_ref[pl.ds(b * SKV, SKV), pl.ds(h * DH, DH)]
                v = v_ref[pl.ds(b * SKV, SKV), pl.ds(h * DH, DH)]
                s = lax.dot_general(
                    q, k, (((1,), (1,)), ((), ())),
                    preferred_element_type=jnp.float32) * SCALE
                w = jnp.exp(jnp.where(mask, s, -1e9))
                ctx = jnp.dot(w, v,
                              preferred_element_type=jnp.float32)
                ctx = ctx / jnp.sum(w, axis=-1, keepdims=True)
                accb = accb + jnp.dot(
                    ctx, wo_s[h * DH:(h + 1) * DH, :],
                    preferred_element_type=jnp.float32)
            acc[pl.ds(b * SQ, SQ), :] = accb

        class Chain:

            def __init__(self, c0, order, send_sems, recv_sems):
                self.c0 = c0
                self.send_sems = send_sems
                self.recv_sems = recv_sems
                s1, s2 = order
                v_first = (m >> s1) & 3
                v_second = (m >> s2) & 3
                base2 = v_first * 256
                keep = base2 + v_second * 64
                self.steps = (
                    ("rs", s1, v_first, 0, 0),
                    ("rs", s2, v_second, base2, base2),
                    ("ag", s2, v_second, base2, keep),
                    ("ag", s1, v_first, 0, base2),
                )

            def stage(self, k, d):
                kind, shift, vlev, base, s_off = self.steps[k]
                size = STEP_SIZES[k]
                cs = pl.ds(self.c0, COLW)
                if kind == "rs":
                    srow = base + (vlev ^ d) * size
                    sbuf[pl.ds((d - 1) * size, size), cs] = (
                        acc[pl.ds(srow, size), cs].astype(COMM))
                elif d == 1:
                    sbuf[pl.ds(0, size), cs] = (
                        acc[pl.ds(s_off, size), cs].astype(COMM))

            def make_rdma(self, k, d):
                kind, shift, vlev, base, s_off = self.steps[k]
                size = STEP_SIZES[k]
                roff = STEP_ROFF[k]
                cs = pl.ds(self.c0, COLW)
                slot = (d - 1) * size if kind == "rs" else 0
                return pltpu.make_async_remote_copy(
                    src_ref=sbuf.at[pl.ds(slot, size), cs],
                    dst_ref=rbuf.at[pl.ds(roff + (d - 1) * size, size), cs],
                    send_sem=self.send_sems.at[k * 3 + d - 1],
                    recv_sem=self.recv_sems.at[k * 3 + d - 1],
                    device_id=(m ^ (d << shift),),
                    device_id_type=pl.DeviceIdType.MESH,
                )

            def send_one(self, k, d):
                self.stage(k, d)
                rdma = self.make_rdma(k, d)
                rdma.start()
                return rdma

            def send_step(self, k, ds=(1, 2, 3)):
                return [self.send_one(k, d) for d in ds]

            def process(self, k):
                kind, shift, vlev, base, _ = self.steps[k]
                size = STEP_SIZES[k]
                roff = STEP_ROFF[k]
                cs = pl.ds(self.c0, COLW)
                if kind == "rs":
                    keep = base + vlev * size
                    total = acc[pl.ds(keep, size), cs]
                    for j in range(3):
                        total = total + rbuf[
                            pl.ds(roff + j * size, size), cs].astype(
                            jnp.float32)
                    acc[pl.ds(keep, size), cs] = total
                else:
                    for d in (1, 2, 3):
                        acc[pl.ds(base + (vlev ^ d) * size, size), cs] = (
                            rbuf[pl.ds(roff + (d - 1) * size, size),
                                 cs].astype(jnp.float32))

        ca = Chain(0, (0, 2), sa_send, sa_recv)
        cb = Chain(COLW, (2, 0), sb_send, sb_recv)

        v0 = m & 3
        v1 = (m >> 2) & 3
        b_other = 1 - (v0 >> 1)
        compute_batch(b_other)
        pl.semaphore_wait(barrier, 6)
        ra = [None] * 4
        rb = [None] * 4
        ra[0] = ca.send_step(0, ds=(2, 3))
        rb[0] = [cb.make_rdma(0, d) for d in (1, 2, 3)]

        def b_early(d):
            def _go():
                cb.stage(0, d)
                rb[0][d - 1].start()
            return _go

        for d in (1, 2, 3):
            in_first = ((v1 ^ d) >> 1) == b_other
            pl.when(in_first)(b_early(d))
        compute_batch(1 - b_other)
        ra[0] += ca.send_step(0, ds=(1,))
        for d in (1, 2, 3):
            in_first = ((v1 ^ d) >> 1) == b_other
            pl.when(jnp.logical_not(in_first))(b_early(d))

        for k in range(4):
            for r in ra[k]:
                r.wait()
            ca.process(k)
            if k < 3:
                ra[k + 1] = ca.send_step(k + 1)
            for r in rb[k]:
                r.wait()
            cb.process(k)
            if k < 3:
                rb[k + 1] = cb.send_step(k + 1)

        for b in range(B):
            out_ref[b, :, :] = acc[b * SQ:(b + 1) * SQ, :]

    return pl.pallas_call(
        body,
        out_shape=jax.ShapeDtypeStruct((B, SQ, D_MODEL), jnp.float32),
        in_specs=[
            pl.BlockSpec(memory_space=pltpu.MemorySpace.VMEM),
            pl.BlockSpec(memory_space=pl.ANY),
            pl.BlockSpec(memory_space=pltpu.MemorySpace.VMEM),
            pl.BlockSpec(memory_space=pltpu.MemorySpace.VMEM),
            pl.BlockSpec(memory_space=pl.ANY),
        ],
        out_specs=pl.BlockSpec(memory_space=pltpu.MemorySpace.VMEM),
        scratch_shapes=[
            pltpu.VMEM((D_MODEL, D_LOC), jnp.float32),
            pltpu.VMEM((D_LOC, D_MODEL), jnp.float32),
            pltpu.VMEM((ROWS, D_MODEL), jnp.float32),
            pltpu.VMEM((768, D_MODEL), COMM),
            pltpu.VMEM((1920, D_MODEL), COMM),
            pltpu.SemaphoreType.DMA((2,)),
            pltpu.SemaphoreType.DMA((12,)),
            pltpu.SemaphoreType.DMA((12,)),
            pltpu.SemaphoreType.DMA((12,)),
            pltpu.SemaphoreType.DMA((12,)),
        ],
        compiler_params=pltpu.CompilerParams(collective_id=0),
    )(x2, Wq, K2, V2, Wo)
